# baseline (speedup 1.0000x reference)
import functools

import jax
import jax.numpy as jnp
from jax.experimental import pallas as pl
from jax.experimental.pallas import tpu as pltpu
from jax.experimental.pallas import tpu_sc as plsc

_D = 384
_COSTS = (0.05, 0.25, 0.6)
_NC, _NS = 2, 16
_NW = _NC * _NS


def _first_min_idx(d2, n_codes):
    m = jnp.min(d2, axis=1, keepdims=True)
    eq = (d2 == m).astype(jnp.float32)
    iota_s = jax.lax.broadcasted_iota(jnp.int32, (n_codes, n_codes), 0)
    w = jax.lax.bitcast_convert_type((127 - iota_s) << 23, jnp.float32)
    se = jax.lax.dot_general(eq, w, (((1,), (0,)), ((), ())),
                             preferred_element_type=jnp.float32)
    ebits = jax.lax.shift_right_logical(
        jax.lax.bitcast_convert_type(se, jnp.int32), 23)
    idx_rep = 127 - ebits
    return idx_rep, m


def _vq_body(x_ref, cb_ref, idx_ref, loss_ref, *, n_codes):
    x = x_ref[...]
    cb = cb_ref[...]
    x2 = jnp.sum(x * x, axis=1, keepdims=True)
    cb2 = jnp.sum(cb * cb, axis=1)[None, :]
    xc = jax.lax.dot_general(x, cb, (((1,), (1,)), ((), ())),
                             preferred_element_type=jnp.float32)
    d2 = x2 - 2.0 * xc + cb2
    idx_rep, m = _first_min_idx(d2, n_codes)
    idx_ref[...] = idx_rep[:, :1]
    s = jnp.sum(m)

    @pl.when(pl.program_id(0) == 0)
    def _init():
        loss_ref[0, 0] = 0.0

    loss_ref[0, 0] += s


def _vq_body_q(x_ref, cb_ref, idx_ref, loss_ref, q_ref, *, n_codes):
    x = x_ref[...]
    cb = cb_ref[...]
    x2 = jnp.sum(x * x, axis=1, keepdims=True)
    cb2 = jnp.sum(cb * cb, axis=1)[None, :]
    xc = jax.lax.dot_general(x, cb, (((1,), (1,)), ((), ())),
                             preferred_element_type=jnp.float32)
    d2 = x2 - 2.0 * xc + cb2
    idx_rep, m = _first_min_idx(d2, n_codes)
    idx_ref[...] = idx_rep[:, :1]
    iota = jax.lax.broadcasted_iota(jnp.int32, d2.shape, 1)
    onehot = (iota == idx_rep).astype(jnp.float32)
    cb_hi = cb.astype(jnp.bfloat16).astype(jnp.float32)
    r1 = cb - cb_hi
    cb_md = r1.astype(jnp.bfloat16).astype(jnp.float32)
    cb_lo = r1 - cb_md
    dn = (((1,), (0,)), ((), ()))
    q_hi = jax.lax.dot_general(onehot, cb_hi, dn,
                               preferred_element_type=jnp.float32)
    q_md = jax.lax.dot_general(onehot, cb_md, dn,
                               preferred_element_type=jnp.float32)
    q_lo = jax.lax.dot_general(onehot, cb_lo, dn,
                               preferred_element_type=jnp.float32)
    q_ref[...] = (q_hi + q_md) + q_lo
    s = jnp.sum(m)

    @pl.when(pl.program_id(0) == 0)
    def _init():
        loss_ref[0, 0] = 0.0

    loss_ref[0, 0] += s


def _vq_level(x_flat, cb, block_rows, with_q):
    n, d = x_flat.shape
    k = cb.shape[0]
    grid = n // block_rows
    out_specs = [
        pl.BlockSpec((block_rows, 1), lambda i: (i, 0)),
        pl.BlockSpec((1, 1), lambda i: (0, 0), memory_space=pltpu.SMEM),
    ]
    out_shape = [
        jax.ShapeDtypeStruct((n, 1), jnp.int32),
        jax.ShapeDtypeStruct((1, 1), jnp.float32),
    ]
    if with_q:
        body = functools.partial(_vq_body_q, n_codes=k)
        out_specs.append(pl.BlockSpec((block_rows, d), lambda i: (i, 0)))
        out_shape.append(jax.ShapeDtypeStruct((n, d), jnp.float32))
    else:
        body = functools.partial(_vq_body, n_codes=k)
    outs = pl.pallas_call(
        body,
        grid=(grid,),
        in_specs=[
            pl.BlockSpec((block_rows, d), lambda i: (i, 0)),
            pl.BlockSpec((k, d), lambda i: (0, 0)),
        ],
        out_specs=out_specs,
        out_shape=out_shape,
    )(x_flat, cb)
    if with_q:
        idx, loss_sum, q = outs
        return idx.reshape(n), loss_sum[0, 0], q
    idx, loss_sum = outs
    return idx.reshape(n), loss_sum[0, 0], None


def _sc_gather(cb, idx, n_rows, chunk):
    rpw = n_rows // _NW
    nchunks = rpw // chunk
    mesh = plsc.VectorSubcoreMesh(
        core_axis_name="c", subcore_axis_name="s",
        num_cores=_NC, num_subcores=_NS)

    @functools.partial(
        pl.kernel,
        out_type=jax.ShapeDtypeStruct((n_rows, _D), jnp.float32),
        mesh=mesh,
        scratch_types=[
            pltpu.VMEM((rpw,), jnp.int32),
            pltpu.VMEM((chunk, _D), jnp.float32),
            pltpu.VMEM((chunk, _D), jnp.float32),
            pltpu.SemaphoreType.DMA,
            pltpu.SemaphoreType.DMA,
        ],
    )
    def gather_kernel(cb_hbm, idx_hbm, out_hbm, idx_v, rows_a, rows_b, gsem, osem):
        wid = jax.lax.axis_index("s") * _NC + jax.lax.axis_index("c")
        base = wid * rpw
        pltpu.sync_copy(idx_hbm.at[pl.ds(base, rpw)], idx_v)
        bufs = (rows_a, rows_b)
        g_descs = [None] * nchunks
        o_descs = [None] * nchunks
        g_descs[0] = pltpu.async_copy(
            cb_hbm.at[idx_v.at[pl.ds(0, chunk)]], bufs[0], gsem)
        for c in range(nchunks):
            if c + 1 < nchunks:
                if c >= 1:
                    o_descs[c - 1].wait()
                g_descs[c + 1] = pltpu.async_copy(
                    cb_hbm.at[idx_v.at[pl.ds((c + 1) * chunk, chunk)]],
                    bufs[(c + 1) % 2], gsem)
            g_descs[c].wait()
            o_descs[c] = pltpu.async_copy(
                bufs[c % 2], out_hbm.at[pl.ds(base + c * chunk, chunk)], osem)
        if nchunks >= 2:
            o_descs[nchunks - 2].wait()
        o_descs[nchunks - 1].wait()

    return gather_kernel(cb, idx)


def kernel(l0, l1, l2, cb0, cb1, cb2):
    levels = ((l0, cb0, 1024, 64, False), (l1, cb1, 1024, 128, False),
              (l2, cb2, 1024, 128, True))
    idxs, qs, sums = [], [], []
    for x, cb, br, chunk, with_q in levels:
        xf = x.reshape(-1, _D)
        idx, s, q = _vq_level(xf, cb, br, with_q)
        if q is None:
            q = _sc_gather(cb, idx, xf.shape[0], chunk)
        idxs.append(idx.reshape(x.shape[:-1]))
        qs.append(q.reshape(x.shape))
        sums.append(s)
    total = (
        (1.0 + _COSTS[0]) * sums[0] / l0.size
        + (1.0 + _COSTS[1]) * sums[1] / l1.size
        + (1.0 + _COSTS[2]) * sums[2] / l2.size
    )
    return (idxs[0], idxs[1], idxs[2], total, qs[0], qs[1], qs[2])


# speedup vs baseline: 1.1155x; 1.1155x over previous
import functools

import jax
import jax.numpy as jnp
from jax.experimental import pallas as pl
from jax.experimental.pallas import tpu as pltpu
from jax.experimental.pallas import tpu_sc as plsc

_D = 384
_COSTS = (0.05, 0.25, 0.6)
_NC, _NS = 2, 16
_NW = _NC * _NS


def _first_min_idx(d2, n_codes):
    m = jnp.min(d2, axis=1, keepdims=True)
    eq = (d2 == m).astype(jnp.float32)
    iota_s = jax.lax.broadcasted_iota(jnp.int32, (n_codes, n_codes), 0)
    w = jax.lax.bitcast_convert_type((127 - iota_s) << 23, jnp.float32)
    se = jax.lax.dot_general(eq, w, (((1,), (0,)), ((), ())),
                             preferred_element_type=jnp.float32)
    ebits = jax.lax.shift_right_logical(
        jax.lax.bitcast_convert_type(se, jnp.int32), 23)
    idx_rep = 127 - ebits
    return idx_rep, m


def _distances(x, cb, n_codes):
    cb2 = jnp.sum(cb * cb, axis=1)[None, :]
    xc = jax.lax.dot_general(x, cb, (((1,), (1,)), ((), ())),
                             preferred_element_type=jnp.float32)
    ones = jnp.ones((_D, n_codes), jnp.float32)
    x2 = jax.lax.dot_general(x * x, ones, (((1,), (0,)), ((), ())),
                             preferred_element_type=jnp.float32)
    return x2 - 2.0 * xc + cb2


def _vq_body(x_ref, cb_ref, idx_ref, loss_ref, *, n_codes):
    x = x_ref[...]
    cb = cb_ref[...]
    d2 = _distances(x, cb, n_codes)
    idx_rep, m = _first_min_idx(d2, n_codes)
    idx_ref[...] = idx_rep[:, 0]
    s = jnp.sum(m)

    @pl.when(pl.program_id(0) == 0)
    def _init():
        loss_ref[0, 0] = 0.0

    loss_ref[0, 0] += s


def _vq_body_q(x_ref, cb_ref, idx_ref, loss_ref, q_ref, *, n_codes):
    x = x_ref[...]
    cb = cb_ref[...]
    d2 = _distances(x, cb, n_codes)
    idx_rep, m = _first_min_idx(d2, n_codes)
    idx_ref[...] = idx_rep[:, 0]
    iota = jax.lax.broadcasted_iota(jnp.int32, d2.shape, 1)
    onehot = (iota == idx_rep).astype(jnp.float32)
    cb_hi = cb.astype(jnp.bfloat16).astype(jnp.float32)
    cb_lo = cb - cb_hi
    dn = (((1,), (0,)), ((), ()))
    q_hi = jax.lax.dot_general(onehot, cb_hi, dn,
                               preferred_element_type=jnp.float32)
    q_lo = jax.lax.dot_general(onehot, cb_lo, dn,
                               preferred_element_type=jnp.float32)
    q_ref[...] = q_hi + q_lo
    s = jnp.sum(m)

    @pl.when(pl.program_id(0) == 0)
    def _init():
        loss_ref[0, 0] = 0.0

    loss_ref[0, 0] += s


def _vq_level(x_flat, cb, block_rows, with_q):
    n, d = x_flat.shape
    k = cb.shape[0]
    grid = n // block_rows
    out_specs = [
        pl.BlockSpec((block_rows,), lambda i: (i,)),
        pl.BlockSpec((1, 1), lambda i: (0, 0), memory_space=pltpu.SMEM),
    ]
    out_shape = [
        jax.ShapeDtypeStruct((n,), jnp.int32),
        jax.ShapeDtypeStruct((1, 1), jnp.float32),
    ]
    if with_q:
        body = functools.partial(_vq_body_q, n_codes=k)
        out_specs.append(pl.BlockSpec((block_rows, d), lambda i: (i, 0)))
        out_shape.append(jax.ShapeDtypeStruct((n, d), jnp.float32))
    else:
        body = functools.partial(_vq_body, n_codes=k)
    outs = pl.pallas_call(
        body,
        grid=(grid,),
        in_specs=[
            pl.BlockSpec((block_rows, d), lambda i: (i, 0)),
            pl.BlockSpec((k, d), lambda i: (0, 0)),
        ],
        out_specs=out_specs,
        out_shape=out_shape,
    )(x_flat, cb)
    if with_q:
        idx, loss_sum, q = outs
        return idx, loss_sum[0, 0], q
    idx, loss_sum = outs
    return idx, loss_sum[0, 0], None


def _sc_gather2(cb_a, idx_a, n_a, chunk_a, cb_b, idx_b, n_b, chunk_b):
    rpw_a, rpw_b = n_a // _NW, n_b // _NW
    mesh = plsc.VectorSubcoreMesh(
        core_axis_name="c", subcore_axis_name="s",
        num_cores=_NC, num_subcores=_NS)
    maxchunk = max(chunk_a, chunk_b)

    @functools.partial(
        pl.kernel,
        out_type=(jax.ShapeDtypeStruct((n_a, _D), jnp.float32),
                  jax.ShapeDtypeStruct((n_b, _D), jnp.float32)),
        mesh=mesh,
        scratch_types=[
            pltpu.VMEM((rpw_a,), jnp.int32),
            pltpu.VMEM((rpw_b,), jnp.int32),
            pltpu.VMEM((maxchunk, _D), jnp.float32),
            pltpu.VMEM((maxchunk, _D), jnp.float32),
            pltpu.SemaphoreType.DMA,
            pltpu.SemaphoreType.DMA,
        ],
    )
    def gather_kernel(cba_hbm, idxa_hbm, cbb_hbm, idxb_hbm, outa_hbm, outb_hbm,
                      idxa_v, idxb_v, rows_x, rows_y, gsem, osem):
        wid = jax.lax.axis_index("s") * _NC + jax.lax.axis_index("c")
        base_a = wid * rpw_a
        base_b = wid * rpw_b
        pltpu.sync_copy(idxa_hbm.at[pl.ds(base_a, rpw_a)], idxa_v)
        pltpu.sync_copy(idxb_hbm.at[pl.ds(base_b, rpw_b)], idxb_v)

        chunks = []
        for c in range(rpw_a // chunk_a):
            chunks.append((cba_hbm, idxa_v, c * chunk_a, outa_hbm,
                           base_a + c * chunk_a, chunk_a))
        for c in range(rpw_b // chunk_b):
            chunks.append((cbb_hbm, idxb_v, c * chunk_b, outb_hbm,
                           base_b + c * chunk_b, chunk_b))
        nch = len(chunks)
        bufs = (rows_x, rows_y)

        def gather(i, buf):
            src, idxv, ioff, _, _, rows = chunks[i]
            return pltpu.async_copy(
                src.at[idxv.at[pl.ds(ioff, rows)]],
                buf.at[pl.ds(0, rows)], gsem)

        def put(i, buf):
            _, _, _, out, ooff, rows = chunks[i]
            return pltpu.async_copy(
                buf.at[pl.ds(0, rows)], out.at[pl.ds(ooff, rows)], osem)

        g_descs = [None] * nch
        o_descs = [None] * nch
        g_descs[0] = gather(0, bufs[0])
        for c in range(nch):
            if c + 1 < nch:
                if c >= 1:
                    o_descs[c - 1].wait()
                g_descs[c + 1] = gather(c + 1, bufs[(c + 1) % 2])
            g_descs[c].wait()
            o_descs[c] = put(c, bufs[c % 2])
        if nch >= 2:
            o_descs[nch - 2].wait()
        o_descs[nch - 1].wait()

    return gather_kernel(cb_a, idx_a, cb_b, idx_b)


def kernel(l0, l1, l2, cb0, cb1, cb2):
    x0 = l0.reshape(-1, _D)
    x1 = l1.reshape(-1, _D)
    x2 = l2.reshape(-1, _D)
    idx0, s0, _ = _vq_level(x0, cb0, 1024, False)
    idx1, s1, _ = _vq_level(x1, cb1, 1024, False)
    idx2, s2, q2 = _vq_level(x2, cb2, 1024, True)
    q0, q1 = _sc_gather2(cb0, idx0, x0.shape[0], 64,
                         cb1, idx1, x1.shape[0], 128)
    total = (
        (1.0 + _COSTS[0]) * s0 / l0.size
        + (1.0 + _COSTS[1]) * s1 / l1.size
        + (1.0 + _COSTS[2]) * s2 / l2.size
    )
    return (idx0.reshape(l0.shape[:-1]), idx1.reshape(l1.shape[:-1]),
            idx2.reshape(l2.shape[:-1]), total,
            q0.reshape(l0.shape), q1.reshape(l1.shape), q2.reshape(l2.shape))


# speedup vs baseline: 1.1727x; 1.0513x over previous
import functools

import jax
import jax.numpy as jnp
from jax.experimental import pallas as pl
from jax.experimental.pallas import tpu as pltpu
from jax.experimental.pallas import tpu_sc as plsc

_D = 384
_COSTS = (0.05, 0.25, 0.6)
_NC, _NS = 2, 16
_NW = _NC * _NS


def _first_min_idx(d2, n_codes):
    m = jnp.min(d2, axis=1, keepdims=True)
    eq = (d2 == m).astype(jnp.float32)
    iota_s = jax.lax.broadcasted_iota(jnp.int32, (n_codes, n_codes), 0)
    w = jax.lax.bitcast_convert_type((127 - iota_s) << 23, jnp.float32)
    se = jax.lax.dot_general(eq, w, (((1,), (0,)), ((), ())),
                             preferred_element_type=jnp.float32)
    ebits = jax.lax.shift_right_logical(
        jax.lax.bitcast_convert_type(se, jnp.int32), 23)
    idx_rep = 127 - ebits
    return idx_rep, m


def _distances(x, cb, n_codes):
    cb2 = jnp.sum(cb * cb, axis=1)[None, :]
    xc = jax.lax.dot_general(x, cb, (((1,), (1,)), ((), ())),
                             preferred_element_type=jnp.float32)
    ones = jnp.ones((_D, n_codes), jnp.float32)
    x2 = jax.lax.dot_general(x * x, ones, (((1,), (0,)), ((), ())),
                             preferred_element_type=jnp.float32)
    return x2 - 2.0 * xc + cb2


def _vq_body(x_ref, cb_ref, idx_ref, loss_ref, *, n_codes):
    x = x_ref[...]
    cb = cb_ref[...]
    d2 = _distances(x, cb, n_codes)
    idx_rep, m = _first_min_idx(d2, n_codes)
    idx_ref[...] = idx_rep[:, 0]
    s = jnp.sum(m)

    @pl.when(pl.program_id(0) == 0)
    def _init():
        loss_ref[0, 0] = 0.0

    loss_ref[0, 0] += s


def _vq_body_q(x_ref, cb_ref, idx_ref, loss_ref, q_ref, *, n_codes):
    x = x_ref[...]
    cb = cb_ref[...]
    d2 = _distances(x, cb, n_codes)
    idx_rep, m = _first_min_idx(d2, n_codes)
    idx_ref[...] = idx_rep[:, 0]
    iota = jax.lax.broadcasted_iota(jnp.int32, d2.shape, 1)
    onehot = (iota == idx_rep).astype(jnp.float32)
    cb_hi = cb.astype(jnp.bfloat16).astype(jnp.float32)
    cb_lo = cb - cb_hi
    dn = (((1,), (0,)), ((), ()))
    q_hi = jax.lax.dot_general(onehot, cb_hi, dn,
                               preferred_element_type=jnp.float32)
    q_lo = jax.lax.dot_general(onehot, cb_lo, dn,
                               preferred_element_type=jnp.float32)
    q_ref[...] = q_hi + q_lo
    s = jnp.sum(m)

    @pl.when(pl.program_id(0) == 0)
    def _init():
        loss_ref[0, 0] = 0.0

    loss_ref[0, 0] += s


def _vq_level(x_flat, cb, block_rows, with_q):
    n, d = x_flat.shape
    k = cb.shape[0]
    grid = n // block_rows
    out_specs = [
        pl.BlockSpec((block_rows,), lambda i: (i,)),
        pl.BlockSpec((1, 1), lambda i: (0, 0), memory_space=pltpu.SMEM),
    ]
    out_shape = [
        jax.ShapeDtypeStruct((n,), jnp.int32),
        jax.ShapeDtypeStruct((1, 1), jnp.float32),
    ]
    if with_q:
        body = functools.partial(_vq_body_q, n_codes=k)
        out_specs.append(pl.BlockSpec((block_rows, d), lambda i: (i, 0)))
        out_shape.append(jax.ShapeDtypeStruct((n, d), jnp.float32))
    else:
        body = functools.partial(_vq_body, n_codes=k)
    outs = pl.pallas_call(
        body,
        grid=(grid,),
        in_specs=[
            pl.BlockSpec((block_rows, d), lambda i: (i, 0)),
            pl.BlockSpec((k, d), lambda i: (0, 0)),
        ],
        out_specs=out_specs,
        out_shape=out_shape,
    )(x_flat, cb)
    if with_q:
        idx, loss_sum, q = outs
        return idx, loss_sum[0, 0], q
    idx, loss_sum = outs
    return idx, loss_sum[0, 0], None


def _sc_gather2(cb_a, idx_a, n_a, chunk_a, cb_b, idx_b, n_b, chunk_b):
    rpw_a, rpw_b = n_a // _NW, n_b // _NW
    mesh = plsc.VectorSubcoreMesh(
        core_axis_name="c", subcore_axis_name="s",
        num_cores=_NC, num_subcores=_NS)
    maxchunk = max(chunk_a, chunk_b)

    @functools.partial(
        pl.kernel,
        out_type=(jax.ShapeDtypeStruct((n_a, _D), jnp.float32),
                  jax.ShapeDtypeStruct((n_b, _D), jnp.float32)),
        mesh=mesh,
        scratch_types=[
            pltpu.VMEM((rpw_a,), jnp.int32),
            pltpu.VMEM((rpw_b,), jnp.int32),
            pltpu.VMEM((maxchunk, _D), jnp.float32),
            pltpu.VMEM((maxchunk, _D), jnp.float32),
            pltpu.SemaphoreType.DMA,
            pltpu.SemaphoreType.DMA,
        ],
    )
    def gather_kernel(cba_hbm, idxa_hbm, cbb_hbm, idxb_hbm, outa_hbm, outb_hbm,
                      idxa_v, idxb_v, rows_x, rows_y, gsem, osem):
        wid = jax.lax.axis_index("s") * _NC + jax.lax.axis_index("c")
        base_a = wid * rpw_a
        base_b = wid * rpw_b
        pltpu.sync_copy(idxa_hbm.at[pl.ds(base_a, rpw_a)], idxa_v)
        pltpu.sync_copy(idxb_hbm.at[pl.ds(base_b, rpw_b)], idxb_v)

        chunks = []
        for c in range(rpw_a // chunk_a):
            chunks.append((cba_hbm, idxa_v, c * chunk_a, outa_hbm,
                           base_a + c * chunk_a, chunk_a))
        for c in range(rpw_b // chunk_b):
            chunks.append((cbb_hbm, idxb_v, c * chunk_b, outb_hbm,
                           base_b + c * chunk_b, chunk_b))
        nch = len(chunks)
        bufs = (rows_x, rows_y)

        def gather(i, buf):
            src, idxv, ioff, _, _, rows = chunks[i]
            return pltpu.async_copy(
                src.at[idxv.at[pl.ds(ioff, rows)]],
                buf.at[pl.ds(0, rows)], gsem)

        def put(i, buf):
            _, _, _, out, ooff, rows = chunks[i]
            return pltpu.async_copy(
                buf.at[pl.ds(0, rows)], out.at[pl.ds(ooff, rows)], osem)

        g_descs = [None] * nch
        o_descs = [None] * nch
        g_descs[0] = gather(0, bufs[0])
        for c in range(nch):
            if c + 1 < nch:
                if c >= 1:
                    o_descs[c - 1].wait()
                g_descs[c + 1] = gather(c + 1, bufs[(c + 1) % 2])
            g_descs[c].wait()
            o_descs[c] = put(c, bufs[c % 2])
        if nch >= 2:
            o_descs[nch - 2].wait()
        o_descs[nch - 1].wait()

    return gather_kernel(cb_a, idx_a, cb_b, idx_b)


def kernel(l0, l1, l2, cb0, cb1, cb2):
    x0 = l0.reshape(-1, _D)
    x1 = l1.reshape(-1, _D)
    x2 = l2.reshape(-1, _D)
    idx0, s0, _ = _vq_level(x0, cb0, 2048, False)
    idx1, s1, _ = _vq_level(x1, cb1, 2048, False)
    idx2, s2, q2 = _vq_level(x2, cb2, 4096, True)
    q0, q1 = _sc_gather2(cb0, idx0, x0.shape[0], 64,
                         cb1, idx1, x1.shape[0], 128)
    total = (
        (1.0 + _COSTS[0]) * s0 / l0.size
        + (1.0 + _COSTS[1]) * s1 / l1.size
        + (1.0 + _COSTS[2]) * s2 / l2.size
    )
    return (idx0.reshape(l0.shape[:-1]), idx1.reshape(l1.shape[:-1]),
            idx2.reshape(l2.shape[:-1]), total,
            q0.reshape(l0.shape), q1.reshape(l1.shape), q2.reshape(l2.shape))


# speedup vs baseline: 1.3036x; 1.1116x over previous
import functools

import jax
import jax.numpy as jnp
from jax.experimental import pallas as pl
from jax.experimental.pallas import tpu as pltpu
from jax.experimental.pallas import tpu_sc as plsc

_D = 384
_COSTS = (0.05, 0.25, 0.6)
_NC, _NS = 2, 16
_NW = _NC * _NS


def _first_min_idx(d2, n_codes):
    m = jnp.min(d2, axis=1, keepdims=True)
    eq = (d2 == m).astype(jnp.float32)
    iota_s = jax.lax.broadcasted_iota(jnp.int32, (n_codes, n_codes), 0)
    w = jax.lax.bitcast_convert_type((127 - iota_s) << 23, jnp.float32)
    se = jax.lax.dot_general(eq, w, (((1,), (0,)), ((), ())),
                             preferred_element_type=jnp.float32)
    ebits = jax.lax.shift_right_logical(
        jax.lax.bitcast_convert_type(se, jnp.int32), 23)
    idx_rep = 127 - ebits
    return idx_rep, m


def _distances(x, cb, n_codes):
    cb2 = jnp.sum(cb * cb, axis=1)[None, :]
    xc = jax.lax.dot_general(x, cb, (((1,), (1,)), ((), ())),
                             preferred_element_type=jnp.float32)
    ones = jnp.ones((_D, n_codes), jnp.float32)
    x2 = jax.lax.dot_general(x * x, ones, (((1,), (0,)), ((), ())),
                             preferred_element_type=jnp.float32)
    return x2 - 2.0 * xc + cb2


def _vq_body(x_ref, cb_ref, idx_ref, loss_ref, *, n_codes):
    x = x_ref[...]
    cb = cb_ref[...]
    d2 = _distances(x, cb, n_codes)
    idx_rep, m = _first_min_idx(d2, n_codes)
    idx_ref[...] = idx_rep[:, 0]
    s = jnp.sum(m)

    @pl.when(pl.program_id(0) == 0)
    def _init():
        loss_ref[0, 0] = 0.0

    loss_ref[0, 0] += s


def _vq_body_q(x_ref, cb_ref, idx_ref, loss_ref, q_ref, *, n_codes):
    x = x_ref[...]
    cb = cb_ref[...]
    d2 = _distances(x, cb, n_codes)
    idx_rep, m = _first_min_idx(d2, n_codes)
    idx_ref[...] = idx_rep[:, 0]
    iota = jax.lax.broadcasted_iota(jnp.int32, d2.shape, 1)
    onehot = (iota == idx_rep).astype(jnp.float32)
    cb_hi = cb.astype(jnp.bfloat16).astype(jnp.float32)
    cb_lo = cb - cb_hi
    dn = (((1,), (0,)), ((), ()))
    q_hi = jax.lax.dot_general(onehot, cb_hi, dn,
                               preferred_element_type=jnp.float32)
    q_lo = jax.lax.dot_general(onehot, cb_lo, dn,
                               preferred_element_type=jnp.float32)
    q_ref[...] = q_hi + q_lo
    s = jnp.sum(m)

    @pl.when(pl.program_id(0) == 0)
    def _init():
        loss_ref[0, 0] = 0.0

    loss_ref[0, 0] += s


def _vq_level(x_flat, cb, block_rows, with_q):
    n, d = x_flat.shape
    k = cb.shape[0]
    grid = n // block_rows
    out_specs = [
        pl.BlockSpec((block_rows,), lambda i: (i,)),
        pl.BlockSpec((1, 1), lambda i: (0, 0), memory_space=pltpu.SMEM),
    ]
    out_shape = [
        jax.ShapeDtypeStruct((n,), jnp.int32),
        jax.ShapeDtypeStruct((1, 1), jnp.float32),
    ]
    if with_q:
        body = functools.partial(_vq_body_q, n_codes=k)
        out_specs.append(pl.BlockSpec((block_rows, d), lambda i: (i, 0)))
        out_shape.append(jax.ShapeDtypeStruct((n, d), jnp.float32))
    else:
        body = functools.partial(_vq_body, n_codes=k)
    outs = pl.pallas_call(
        body,
        grid=(grid,),
        in_specs=[
            pl.BlockSpec((block_rows, d), lambda i: (i, 0)),
            pl.BlockSpec((k, d), lambda i: (0, 0)),
        ],
        out_specs=out_specs,
        out_shape=out_shape,
    )(x_flat, cb)
    if with_q:
        idx, loss_sum, q = outs
        return idx, loss_sum[0, 0], q
    idx, loss_sum = outs
    return idx, loss_sum[0, 0], None


def _sc_gather(cb, idx, n_rows, chunk):
    rpw = n_rows // _NW
    nchunks = rpw // chunk
    mesh = plsc.VectorSubcoreMesh(
        core_axis_name="c", subcore_axis_name="s",
        num_cores=_NC, num_subcores=_NS)

    @functools.partial(
        pl.kernel,
        out_type=jax.ShapeDtypeStruct((n_rows, _D), jnp.float32),
        mesh=mesh,
        scratch_types=[
            pltpu.VMEM((rpw,), jnp.int32),
            pltpu.VMEM((chunk, _D), jnp.float32),
            pltpu.VMEM((chunk, _D), jnp.float32),
            pltpu.SemaphoreType.DMA,
            pltpu.SemaphoreType.DMA,
        ],
    )
    def gather_kernel(cb_hbm, idx_hbm, out_hbm, idx_v, rows_a, rows_b, gsem, osem):
        wid = jax.lax.axis_index("s") * _NC + jax.lax.axis_index("c")
        base = wid * rpw
        pltpu.sync_copy(idx_hbm.at[pl.ds(base, rpw)], idx_v)
        bufs = (rows_a, rows_b)
        g_descs = [None] * nchunks
        o_descs = [None] * nchunks
        g_descs[0] = pltpu.async_copy(
            cb_hbm.at[idx_v.at[pl.ds(0, chunk)]], bufs[0], gsem)
        for c in range(nchunks):
            if c + 1 < nchunks:
                if c >= 1:
                    o_descs[c - 1].wait()
                g_descs[c + 1] = pltpu.async_copy(
                    cb_hbm.at[idx_v.at[pl.ds((c + 1) * chunk, chunk)]],
                    bufs[(c + 1) % 2], gsem)
            g_descs[c].wait()
            o_descs[c] = pltpu.async_copy(
                bufs[c % 2], out_hbm.at[pl.ds(base + c * chunk, chunk)], osem)
        if nchunks >= 2:
            o_descs[nchunks - 2].wait()
        o_descs[nchunks - 1].wait()

    return gather_kernel(cb, idx)


def _sc_gather2(cb_a, idx_a, n_a, chunk_a, cb_b, idx_b, n_b, chunk_b):
    rpw_a, rpw_b = n_a // _NW, n_b // _NW
    mesh = plsc.VectorSubcoreMesh(
        core_axis_name="c", subcore_axis_name="s",
        num_cores=_NC, num_subcores=_NS)
    maxchunk = max(chunk_a, chunk_b)

    @functools.partial(
        pl.kernel,
        out_type=(jax.ShapeDtypeStruct((n_a, _D), jnp.float32),
                  jax.ShapeDtypeStruct((n_b, _D), jnp.float32)),
        mesh=mesh,
        scratch_types=[
            pltpu.VMEM((rpw_a,), jnp.int32),
            pltpu.VMEM((rpw_b,), jnp.int32),
            pltpu.VMEM((maxchunk, _D), jnp.float32),
            pltpu.VMEM((maxchunk, _D), jnp.float32),
            pltpu.SemaphoreType.DMA,
            pltpu.SemaphoreType.DMA,
        ],
    )
    def gather_kernel(cba_hbm, idxa_hbm, cbb_hbm, idxb_hbm, outa_hbm, outb_hbm,
                      idxa_v, idxb_v, rows_x, rows_y, gsem, osem):
        wid = jax.lax.axis_index("s") * _NC + jax.lax.axis_index("c")
        base_a = wid * rpw_a
        base_b = wid * rpw_b
        pltpu.sync_copy(idxa_hbm.at[pl.ds(base_a, rpw_a)], idxa_v)
        pltpu.sync_copy(idxb_hbm.at[pl.ds(base_b, rpw_b)], idxb_v)

        chunks = []
        for c in range(rpw_a // chunk_a):
            chunks.append((cba_hbm, idxa_v, c * chunk_a, outa_hbm,
                           base_a + c * chunk_a, chunk_a))
        for c in range(rpw_b // chunk_b):
            chunks.append((cbb_hbm, idxb_v, c * chunk_b, outb_hbm,
                           base_b + c * chunk_b, chunk_b))
        nch = len(chunks)
        bufs = (rows_x, rows_y)

        def gather(i, buf):
            src, idxv, ioff, _, _, rows = chunks[i]
            return pltpu.async_copy(
                src.at[idxv.at[pl.ds(ioff, rows)]],
                buf.at[pl.ds(0, rows)], gsem)

        def put(i, buf):
            _, _, _, out, ooff, rows = chunks[i]
            return pltpu.async_copy(
                buf.at[pl.ds(0, rows)], out.at[pl.ds(ooff, rows)], osem)

        g_descs = [None] * nch
        o_descs = [None] * nch
        g_descs[0] = gather(0, bufs[0])
        for c in range(nch):
            if c + 1 < nch:
                if c >= 1:
                    o_descs[c - 1].wait()
                g_descs[c + 1] = gather(c + 1, bufs[(c + 1) % 2])
            g_descs[c].wait()
            o_descs[c] = put(c, bufs[c % 2])
        if nch >= 2:
            o_descs[nch - 2].wait()
        o_descs[nch - 1].wait()

    return gather_kernel(cb_a, idx_a, cb_b, idx_b)


def kernel(l0, l1, l2, cb0, cb1, cb2):
    x0 = l0.reshape(-1, _D)
    x1 = l1.reshape(-1, _D)
    x2 = l2.reshape(-1, _D)
    idx0, s0, _ = _vq_level(x0, cb0, 2048, False)
    idx1, s1, q1 = _vq_level(x1, cb1, 2048, True)
    idx2, s2, q2 = _vq_level(x2, cb2, 2048, True)
    q0 = _sc_gather(cb0, idx0, x0.shape[0], 64)
    total = (
        (1.0 + _COSTS[0]) * s0 / l0.size
        + (1.0 + _COSTS[1]) * s1 / l1.size
        + (1.0 + _COSTS[2]) * s2 / l2.size
    )
    return (idx0.reshape(l0.shape[:-1]), idx1.reshape(l1.shape[:-1]),
            idx2.reshape(l2.shape[:-1]), total,
            q0.reshape(l0.shape), q1.reshape(l1.shape), q2.reshape(l2.shape))


# speedup vs baseline: 1.6871x; 1.2942x over previous
import functools

import jax
import jax.numpy as jnp
from jax.experimental import pallas as pl
from jax.experimental.pallas import tpu as pltpu
from jax.experimental.pallas import tpu_sc as plsc

_D = 384
_COSTS = (0.05, 0.25, 0.6)
_NC, _NS = 2, 16
_NW = _NC * _NS


def _first_min_idx(d2, n_codes):
    m = jnp.min(d2, axis=1, keepdims=True)
    eq = (d2 == m).astype(jnp.float32)
    iota_s = jax.lax.broadcasted_iota(jnp.int32, (n_codes, n_codes), 0)
    w = jax.lax.bitcast_convert_type((127 - iota_s) << 23, jnp.float32)
    se = jax.lax.dot_general(eq, w, (((1,), (0,)), ((), ())),
                             preferred_element_type=jnp.float32)
    ebits = jax.lax.shift_right_logical(
        jax.lax.bitcast_convert_type(se, jnp.int32), 23)
    idx_rep = 127 - ebits
    return idx_rep, m


def _distances(x, cb, n_codes):
    cb2 = jnp.sum(cb * cb, axis=1)[None, :]
    xc = jax.lax.dot_general(x, cb, (((1,), (1,)), ((), ())),
                             preferred_element_type=jnp.float32)
    ones = jnp.ones((_D, n_codes), jnp.float32)
    x2 = jax.lax.dot_general(x * x, ones, (((1,), (0,)), ((), ())),
                             preferred_element_type=jnp.float32)
    return x2 - 2.0 * xc + cb2


def _vq_body(x_ref, cb_ref, idx_ref, loss_ref, *, n_codes):
    x = x_ref[...]
    cb = cb_ref[...]
    d2 = _distances(x, cb, n_codes)
    idx_rep, m = _first_min_idx(d2, n_codes)
    idx_ref[...] = idx_rep[:, 0]
    s = jnp.sum(m)

    @pl.when(pl.program_id(0) == 0)
    def _init():
        loss_ref[0, 0] = 0.0

    loss_ref[0, 0] += s


def _vq_body_q(x_ref, cb_ref, idx_ref, loss_ref, q_ref, *, n_codes):
    x = x_ref[...]
    cb = cb_ref[...]
    d2 = _distances(x, cb, n_codes)
    idx_rep, m = _first_min_idx(d2, n_codes)
    idx_ref[...] = idx_rep[:, 0]
    iota = jax.lax.broadcasted_iota(jnp.int32, d2.shape, 1)
    onehot = (iota == idx_rep).astype(jnp.float32)
    cb_hi = cb.astype(jnp.bfloat16).astype(jnp.float32)
    cb_lo = cb - cb_hi
    dn = (((1,), (0,)), ((), ()))
    q_hi = jax.lax.dot_general(onehot, cb_hi, dn,
                               preferred_element_type=jnp.float32)
    q_lo = jax.lax.dot_general(onehot, cb_lo, dn,
                               preferred_element_type=jnp.float32)
    q_ref[...] = q_hi + q_lo
    s = jnp.sum(m)

    @pl.when(pl.program_id(0) == 0)
    def _init():
        loss_ref[0, 0] = 0.0

    loss_ref[0, 0] += s


def _vq_level(x_flat, cb, block_rows, with_q):
    n, d = x_flat.shape
    k = cb.shape[0]
    grid = n // block_rows
    out_specs = [
        pl.BlockSpec((block_rows,), lambda i: (i,)),
        pl.BlockSpec((1, 1), lambda i: (0, 0), memory_space=pltpu.SMEM),
    ]
    out_shape = [
        jax.ShapeDtypeStruct((n,), jnp.int32),
        jax.ShapeDtypeStruct((1, 1), jnp.float32),
    ]
    if with_q:
        body = functools.partial(_vq_body_q, n_codes=k)
        out_specs.append(pl.BlockSpec((block_rows, d), lambda i: (i, 0)))
        out_shape.append(jax.ShapeDtypeStruct((n, d), jnp.float32))
    else:
        body = functools.partial(_vq_body, n_codes=k)
    outs = pl.pallas_call(
        body,
        grid=(grid,),
        in_specs=[
            pl.BlockSpec((block_rows, d), lambda i: (i, 0)),
            pl.BlockSpec((k, d), lambda i: (0, 0)),
        ],
        out_specs=out_specs,
        out_shape=out_shape,
    )(x_flat, cb)
    if with_q:
        idx, loss_sum, q = outs
        return idx, loss_sum[0, 0], q
    idx, loss_sum = outs
    return idx, loss_sum[0, 0], None


def _sc_gather(cb, idx, n_rows, chunk):
    rpw = n_rows // _NW
    nchunks = rpw // chunk
    mesh = plsc.VectorSubcoreMesh(
        core_axis_name="c", subcore_axis_name="s",
        num_cores=_NC, num_subcores=_NS)

    @functools.partial(
        pl.kernel,
        out_type=jax.ShapeDtypeStruct((n_rows, _D), jnp.float32),
        mesh=mesh,
        scratch_types=[
            pltpu.VMEM((rpw,), jnp.int32),
            pltpu.VMEM((chunk, _D), jnp.float32),
            pltpu.VMEM((chunk, _D), jnp.float32),
            pltpu.SemaphoreType.DMA,
            pltpu.SemaphoreType.DMA,
        ],
    )
    def gather_kernel(cb_hbm, idx_hbm, out_hbm, idx_v, rows_a, rows_b, gsem, osem):
        wid = jax.lax.axis_index("s") * _NC + jax.lax.axis_index("c")
        base = wid * rpw
        pltpu.sync_copy(idx_hbm.at[pl.ds(base, rpw)], idx_v)
        bufs = (rows_a, rows_b)
        g_descs = [None] * nchunks
        o_descs = [None] * nchunks
        g_descs[0] = pltpu.async_copy(
            cb_hbm.at[idx_v.at[pl.ds(0, chunk)]], bufs[0], gsem)
        for c in range(nchunks):
            if c + 1 < nchunks:
                if c >= 1:
                    o_descs[c - 1].wait()
                g_descs[c + 1] = pltpu.async_copy(
                    cb_hbm.at[idx_v.at[pl.ds((c + 1) * chunk, chunk)]],
                    bufs[(c + 1) % 2], gsem)
            g_descs[c].wait()
            o_descs[c] = pltpu.async_copy(
                bufs[c % 2], out_hbm.at[pl.ds(base + c * chunk, chunk)], osem)
        if nchunks >= 2:
            o_descs[nchunks - 2].wait()
        o_descs[nchunks - 1].wait()

    return gather_kernel(cb, idx)


def _sc_gather2(cb_a, idx_a, n_a, chunk_a, cb_b, idx_b, n_b, chunk_b):
    rpw_a, rpw_b = n_a // _NW, n_b // _NW
    mesh = plsc.VectorSubcoreMesh(
        core_axis_name="c", subcore_axis_name="s",
        num_cores=_NC, num_subcores=_NS)
    maxchunk = max(chunk_a, chunk_b)

    @functools.partial(
        pl.kernel,
        out_type=(jax.ShapeDtypeStruct((n_a, _D), jnp.float32),
                  jax.ShapeDtypeStruct((n_b, _D), jnp.float32)),
        mesh=mesh,
        scratch_types=[
            pltpu.VMEM((rpw_a,), jnp.int32),
            pltpu.VMEM((rpw_b,), jnp.int32),
            pltpu.VMEM((maxchunk, _D), jnp.float32),
            pltpu.VMEM((maxchunk, _D), jnp.float32),
            pltpu.SemaphoreType.DMA,
            pltpu.SemaphoreType.DMA,
        ],
    )
    def gather_kernel(cba_hbm, idxa_hbm, cbb_hbm, idxb_hbm, outa_hbm, outb_hbm,
                      idxa_v, idxb_v, rows_x, rows_y, gsem, osem):
        wid = jax.lax.axis_index("s") * _NC + jax.lax.axis_index("c")
        base_a = wid * rpw_a
        base_b = wid * rpw_b
        pltpu.sync_copy(idxa_hbm.at[pl.ds(base_a, rpw_a)], idxa_v)
        pltpu.sync_copy(idxb_hbm.at[pl.ds(base_b, rpw_b)], idxb_v)

        chunks = []
        for c in range(rpw_a // chunk_a):
            chunks.append((cba_hbm, idxa_v, c * chunk_a, outa_hbm,
                           base_a + c * chunk_a, chunk_a))
        for c in range(rpw_b // chunk_b):
            chunks.append((cbb_hbm, idxb_v, c * chunk_b, outb_hbm,
                           base_b + c * chunk_b, chunk_b))
        nch = len(chunks)
        bufs = (rows_x, rows_y)

        def gather(i, buf):
            src, idxv, ioff, _, _, rows = chunks[i]
            return pltpu.async_copy(
                src.at[idxv.at[pl.ds(ioff, rows)]],
                buf.at[pl.ds(0, rows)], gsem)

        def put(i, buf):
            _, _, _, out, ooff, rows = chunks[i]
            return pltpu.async_copy(
                buf.at[pl.ds(0, rows)], out.at[pl.ds(ooff, rows)], osem)

        g_descs = [None] * nch
        o_descs = [None] * nch
        g_descs[0] = gather(0, bufs[0])
        for c in range(nch):
            if c + 1 < nch:
                if c >= 1:
                    o_descs[c - 1].wait()
                g_descs[c + 1] = gather(c + 1, bufs[(c + 1) % 2])
            g_descs[c].wait()
            o_descs[c] = put(c, bufs[c % 2])
        if nch >= 2:
            o_descs[nch - 2].wait()
        o_descs[nch - 1].wait()

    return gather_kernel(cb_a, idx_a, cb_b, idx_b)


def kernel(l0, l1, l2, cb0, cb1, cb2):
    x0 = l0.reshape(-1, _D)
    x1 = l1.reshape(-1, _D)
    x2 = l2.reshape(-1, _D)
    idx0, s0, q0 = _vq_level(x0, cb0, 2048, True)
    idx1, s1, q1 = _vq_level(x1, cb1, 2048, True)
    idx2, s2, q2 = _vq_level(x2, cb2, 2048, True)
    total = (
        (1.0 + _COSTS[0]) * s0 / l0.size
        + (1.0 + _COSTS[1]) * s1 / l1.size
        + (1.0 + _COSTS[2]) * s2 / l2.size
    )
    return (idx0.reshape(l0.shape[:-1]), idx1.reshape(l1.shape[:-1]),
            idx2.reshape(l2.shape[:-1]), total,
            q0.reshape(l0.shape), q1.reshape(l1.shape), q2.reshape(l2.shape))


# speedup vs baseline: 1.7119x; 1.0147x over previous
import functools

import jax
import jax.numpy as jnp
from jax.experimental import pallas as pl
from jax.experimental.pallas import tpu as pltpu
from jax.experimental.pallas import tpu_sc as plsc

_D = 384
_COSTS = (0.05, 0.25, 0.6)
_NC, _NS = 2, 16
_NW = _NC * _NS


def _first_min_idx(d2, n_codes):
    m = jnp.min(d2, axis=1, keepdims=True)
    eq = (d2 == m).astype(jnp.float32)
    iota_s = jax.lax.broadcasted_iota(jnp.int32, (n_codes, n_codes), 0)
    w = jax.lax.bitcast_convert_type((127 - iota_s) << 23, jnp.float32)
    se = jax.lax.dot_general(eq, w, (((1,), (0,)), ((), ())),
                             preferred_element_type=jnp.float32)
    ebits = jax.lax.shift_right_logical(
        jax.lax.bitcast_convert_type(se, jnp.int32), 23)
    idx_rep = 127 - ebits
    return idx_rep, m


def _distances(x, cb, n_codes):
    cb2 = jnp.sum(cb * cb, axis=1)[None, :]
    xc = jax.lax.dot_general(x, cb, (((1,), (1,)), ((), ())),
                             preferred_element_type=jnp.float32)
    ones = jnp.ones((_D, n_codes), jnp.float32)
    x2 = jax.lax.dot_general(x * x, ones, (((1,), (0,)), ((), ())),
                             preferred_element_type=jnp.float32)
    return x2 - 2.0 * xc + cb2


def _vq_body(x_ref, cb_ref, idx_ref, loss_ref, *, n_codes):
    x = x_ref[...]
    cb = cb_ref[...]
    d2 = _distances(x, cb, n_codes)
    idx_rep, m = _first_min_idx(d2, n_codes)
    idx_ref[...] = idx_rep[:, 0]
    s = jnp.sum(m)

    @pl.when(pl.program_id(0) == 0)
    def _init():
        loss_ref[0, 0] = 0.0

    loss_ref[0, 0] += s


def _vq_body_q(x_ref, cb_ref, idx_ref, loss_ref, q_ref, *, n_codes):
    x = x_ref[...]
    cb = cb_ref[...]
    d2 = _distances(x, cb, n_codes)
    idx_rep, m = _first_min_idx(d2, n_codes)
    idx_ref[...] = idx_rep[:, 0]
    iota = jax.lax.broadcasted_iota(jnp.int32, d2.shape, 1)
    onehot = (iota == idx_rep).astype(jnp.float32)
    cb_hi = cb.astype(jnp.bfloat16).astype(jnp.float32)
    cb_lo = cb - cb_hi
    dn = (((1,), (0,)), ((), ()))
    q_hi = jax.lax.dot_general(onehot, cb_hi, dn,
                               preferred_element_type=jnp.float32)
    q_lo = jax.lax.dot_general(onehot, cb_lo, dn,
                               preferred_element_type=jnp.float32)
    q_ref[...] = q_hi + q_lo
    s = jnp.sum(m)

    @pl.when(pl.program_id(0) == 0)
    def _init():
        loss_ref[0, 0] = 0.0

    loss_ref[0, 0] += s


def _vq_level(x_flat, cb, block_rows, with_q):
    n, d = x_flat.shape
    k = cb.shape[0]
    grid = n // block_rows
    out_specs = [
        pl.BlockSpec((block_rows,), lambda i: (i,)),
        pl.BlockSpec((1, 1), lambda i: (0, 0), memory_space=pltpu.SMEM),
    ]
    out_shape = [
        jax.ShapeDtypeStruct((n,), jnp.int32),
        jax.ShapeDtypeStruct((1, 1), jnp.float32),
    ]
    if with_q:
        body = functools.partial(_vq_body_q, n_codes=k)
        out_specs.append(pl.BlockSpec((block_rows, d), lambda i: (i, 0)))
        out_shape.append(jax.ShapeDtypeStruct((n, d), jnp.float32))
    else:
        body = functools.partial(_vq_body, n_codes=k)
    outs = pl.pallas_call(
        body,
        grid=(grid,),
        in_specs=[
            pl.BlockSpec((block_rows, d), lambda i: (i, 0)),
            pl.BlockSpec((k, d), lambda i: (0, 0)),
        ],
        out_specs=out_specs,
        out_shape=out_shape,
    )(x_flat, cb)
    if with_q:
        idx, loss_sum, q = outs
        return idx, loss_sum[0, 0], q
    idx, loss_sum = outs
    return idx, loss_sum[0, 0], None


def _sc_gather(cb, idx, n_rows, chunk):
    rpw = n_rows // _NW
    nchunks = rpw // chunk
    mesh = plsc.VectorSubcoreMesh(
        core_axis_name="c", subcore_axis_name="s",
        num_cores=_NC, num_subcores=_NS)

    @functools.partial(
        pl.kernel,
        out_type=jax.ShapeDtypeStruct((n_rows, _D), jnp.float32),
        mesh=mesh,
        scratch_types=[
            pltpu.VMEM((rpw,), jnp.int32),
            pltpu.VMEM((chunk, _D), jnp.float32),
            pltpu.VMEM((chunk, _D), jnp.float32),
            pltpu.SemaphoreType.DMA,
            pltpu.SemaphoreType.DMA,
        ],
    )
    def gather_kernel(cb_hbm, idx_hbm, out_hbm, idx_v, rows_a, rows_b, gsem, osem):
        wid = jax.lax.axis_index("s") * _NC + jax.lax.axis_index("c")
        base = wid * rpw
        pltpu.sync_copy(idx_hbm.at[pl.ds(base, rpw)], idx_v)
        bufs = (rows_a, rows_b)
        g_descs = [None] * nchunks
        o_descs = [None] * nchunks
        g_descs[0] = pltpu.async_copy(
            cb_hbm.at[idx_v.at[pl.ds(0, chunk)]], bufs[0], gsem)
        for c in range(nchunks):
            if c + 1 < nchunks:
                if c >= 1:
                    o_descs[c - 1].wait()
                g_descs[c + 1] = pltpu.async_copy(
                    cb_hbm.at[idx_v.at[pl.ds((c + 1) * chunk, chunk)]],
                    bufs[(c + 1) % 2], gsem)
            g_descs[c].wait()
            o_descs[c] = pltpu.async_copy(
                bufs[c % 2], out_hbm.at[pl.ds(base + c * chunk, chunk)], osem)
        if nchunks >= 2:
            o_descs[nchunks - 2].wait()
        o_descs[nchunks - 1].wait()

    return gather_kernel(cb, idx)


def _sc_gather2(cb_a, idx_a, n_a, chunk_a, cb_b, idx_b, n_b, chunk_b):
    rpw_a, rpw_b = n_a // _NW, n_b // _NW
    mesh = plsc.VectorSubcoreMesh(
        core_axis_name="c", subcore_axis_name="s",
        num_cores=_NC, num_subcores=_NS)
    maxchunk = max(chunk_a, chunk_b)

    @functools.partial(
        pl.kernel,
        out_type=(jax.ShapeDtypeStruct((n_a, _D), jnp.float32),
                  jax.ShapeDtypeStruct((n_b, _D), jnp.float32)),
        mesh=mesh,
        scratch_types=[
            pltpu.VMEM((rpw_a,), jnp.int32),
            pltpu.VMEM((rpw_b,), jnp.int32),
            pltpu.VMEM((maxchunk, _D), jnp.float32),
            pltpu.VMEM((maxchunk, _D), jnp.float32),
            pltpu.SemaphoreType.DMA,
            pltpu.SemaphoreType.DMA,
        ],
    )
    def gather_kernel(cba_hbm, idxa_hbm, cbb_hbm, idxb_hbm, outa_hbm, outb_hbm,
                      idxa_v, idxb_v, rows_x, rows_y, gsem, osem):
        wid = jax.lax.axis_index("s") * _NC + jax.lax.axis_index("c")
        base_a = wid * rpw_a
        base_b = wid * rpw_b
        pltpu.sync_copy(idxa_hbm.at[pl.ds(base_a, rpw_a)], idxa_v)
        pltpu.sync_copy(idxb_hbm.at[pl.ds(base_b, rpw_b)], idxb_v)

        chunks = []
        for c in range(rpw_a // chunk_a):
            chunks.append((cba_hbm, idxa_v, c * chunk_a, outa_hbm,
                           base_a + c * chunk_a, chunk_a))
        for c in range(rpw_b // chunk_b):
            chunks.append((cbb_hbm, idxb_v, c * chunk_b, outb_hbm,
                           base_b + c * chunk_b, chunk_b))
        nch = len(chunks)
        bufs = (rows_x, rows_y)

        def gather(i, buf):
            src, idxv, ioff, _, _, rows = chunks[i]
            return pltpu.async_copy(
                src.at[idxv.at[pl.ds(ioff, rows)]],
                buf.at[pl.ds(0, rows)], gsem)

        def put(i, buf):
            _, _, _, out, ooff, rows = chunks[i]
            return pltpu.async_copy(
                buf.at[pl.ds(0, rows)], out.at[pl.ds(ooff, rows)], osem)

        g_descs = [None] * nch
        o_descs = [None] * nch
        g_descs[0] = gather(0, bufs[0])
        for c in range(nch):
            if c + 1 < nch:
                if c >= 1:
                    o_descs[c - 1].wait()
                g_descs[c + 1] = gather(c + 1, bufs[(c + 1) % 2])
            g_descs[c].wait()
            o_descs[c] = put(c, bufs[c % 2])
        if nch >= 2:
            o_descs[nch - 2].wait()
        o_descs[nch - 1].wait()

    return gather_kernel(cb_a, idx_a, cb_b, idx_b)


def kernel(l0, l1, l2, cb0, cb1, cb2):
    x0 = l0.reshape(-1, _D)
    x1 = l1.reshape(-1, _D)
    x2 = l2.reshape(-1, _D)
    idx0, s0, q0 = _vq_level(x0, cb0, 2048, True)
    idx1, s1, q1 = _vq_level(x1, cb1, 2048, True)
    idx2, s2, q2 = _vq_level(x2, cb2, 4096, True)
    total = (
        (1.0 + _COSTS[0]) * s0 / l0.size
        + (1.0 + _COSTS[1]) * s1 / l1.size
        + (1.0 + _COSTS[2]) * s2 / l2.size
    )
    return (idx0.reshape(l0.shape[:-1]), idx1.reshape(l1.shape[:-1]),
            idx2.reshape(l2.shape[:-1]), total,
            q0.reshape(l0.shape), q1.reshape(l1.shape), q2.reshape(l2.shape))
